# baseline (speedup 1.0000x reference)
import functools

import jax
import jax.numpy as jnp
from jax import lax
from jax.experimental import pallas as pl
from jax.experimental.pallas import tpu as pltpu
from jax.experimental.pallas import tpu_sc as plsc

L = 16
NW = 32
CH = 128


def _spline_body(W, ND, bpw, nch, u_hbm, grid_hbm, out_hbm,
                 u_v, idx_v, w_v, rows_v, out_v, out_t, sem0, sem1):
    nc = 2
    wid = lax.axis_index("s") * nc + lax.axis_index("c")
    base = wid * bpw

    pltpu.sync_copy(u_hbm.at[pl.ds(base, bpw)], u_v)

    def wcomp(g, carry):
        sl = pl.ds(g * L, L)
        uu = jnp.clip(u_v[sl], 0.0, 1.0)
        x = uu * jnp.float32(W - 1)
        idx = jnp.minimum(x.astype(jnp.int32), W - 2)
        t = x - idx.astype(jnp.float32)
        t2 = t * t
        t3 = t2 * t
        sixth = jnp.float32(1.0 / 6.0)
        w0 = sixth * (-t3 + 3.0 * t2 - 3.0 * t + 1.0)
        w1 = sixth * (3.0 * t3 - 6.0 * t2 + 4.0)
        w2 = 0.5 * (-t3 + t2 + t) + sixth
        w3 = sixth * t3
        is_lo = idx == 0
        is_hi = idx == W - 2
        v0 = jnp.where(is_lo, 2.0 * w0, w0)
        v1 = jnp.where(is_hi, w1 - w3, w1)
        v2 = w2 + jnp.where(is_lo, -w0, jnp.where(is_hi, w3, 0.0))
        v3 = w3
        idx_v[0, sl] = jnp.maximum(idx - 1, 0)
        idx_v[1, sl] = idx
        idx_v[2, sl] = idx + 1
        idx_v[3, sl] = jnp.minimum(idx + 2, W - 1)
        w_v[0, sl] = v0
        w_v[1, sl] = v1
        w_v[2, sl] = v2
        w_v[3, sl] = v3
        return carry

    lax.fori_loop(0, bpw // L, wcomp, 0, unroll=2)

    sems = (sem0, sem1)
    lane = lax.iota(jnp.int32, L)
    lanep = lane * (ND + 1)

    def fire(c):
        return [
            pltpu.async_copy(
                grid_hbm.at[idx_v.at[k, pl.ds(c * CH, CH)]],
                rows_v.at[c % 2, k], sems[c % 2])
            for k in range(4)
        ]

    inflight = fire(0)
    for c in range(nch):
        for cp in inflight:
            cp.wait()
        if c + 1 < nch:
            inflight = fire(c + 1)

        def scomp(g2, carry, c=c):
            wsl = pl.ds(c * CH + g2 * L, L)
            a0 = w_v[0, wsl]
            a1 = w_v[1, wsl]
            a2 = w_v[2, wsl]
            a3 = w_v[3, wsl]
            for j in range(L):
                b = g2 * L + j
                s0, s1, s2, s3 = a0[j], a1[j], a2[j], a3[j]
                ob = b * (ND + 1)
                for jj in range(ND // L):
                    sl = pl.ds(jj * L, L)
                    acc = (rows_v[c % 2, 0, b, sl] * s0
                           + rows_v[c % 2, 1, b, sl] * s1
                           + rows_v[c % 2, 2, b, sl] * s2
                           + rows_v[c % 2, 3, b, sl] * s3)
                    out_v[pl.ds(ob + jj * L, L)] = acc
            return carry

        lax.fori_loop(0, CH // L, scomp, 0)

        def tcomp(i, carry):
            ch = lax.shift_right_arithmetic(i, 3)
            b0 = lax.bitwise_and(i, 7) * L
            pos = lanep + (b0 * (ND + 1) + ch)
            out_t[ch, pl.ds(b0, L)] = plsc.load_gather(out_v, [pos])
            return carry

        lax.fori_loop(0, (ND * CH) // L, tcomp, 0)
        rb = wid * nch + c
        for cb8 in range(ND // 8):
            pltpu.sync_copy(
                out_t.at[pl.ds(cb8 * 8, 8), :],
                out_hbm.at[pl.ds((cb8 * (NW * nch) + rb) * 8, 8), :])


def kernel(u, grid):
    B = u.shape[0]
    W, ND = grid.shape
    bpw = B // NW
    nch = bpw // CH
    mesh = plsc.VectorSubcoreMesh(core_axis_name="c", subcore_axis_name="s")
    body = functools.partial(_spline_body, W, ND, bpw, nch)
    f = pl.kernel(
        body,
        mesh=mesh,
        out_type=jax.ShapeDtypeStruct((B * ND // CH, CH), jnp.float32),
        scratch_types=[
            pltpu.VMEM((bpw,), jnp.float32),
            pltpu.VMEM((4, bpw), jnp.int32),
            pltpu.VMEM((4, bpw), jnp.float32),
            pltpu.VMEM((2, 4, CH, ND), jnp.float32),
            pltpu.VMEM((CH * (ND + 1),), jnp.float32),
            pltpu.VMEM((ND, CH), jnp.float32),
            pltpu.SemaphoreType.DMA,
            pltpu.SemaphoreType.DMA,
        ],
        compiler_params=pltpu.CompilerParams(
            use_tc_tiling_on_sc=False, needs_layout_passes=False),
    )
    out2 = f(u, grid)
    return (out2.reshape(ND // 8, B // CH, 8, CH)
            .transpose(1, 3, 0, 2).reshape(B, ND))


# speedup vs baseline: 1.0746x; 1.0746x over previous
import functools

import jax
import jax.numpy as jnp
from jax import lax
from jax.experimental import pallas as pl
from jax.experimental.pallas import tpu as pltpu
from jax.experimental.pallas import tpu_sc as plsc

L = 16
NW = 32
CH = 128


def _spline_body(W, ND, bpw, nch, u_hbm, grid_hbm, out_hbm,
                 u_v, idx_v, w_v, rows_v, out_v, sem0, sem1):
    nc = 2
    wid = lax.axis_index("s") * nc + lax.axis_index("c")
    base = wid * bpw

    pltpu.sync_copy(u_hbm.at[pl.ds(base, bpw)], u_v)

    def wcomp(g, carry):
        sl = pl.ds(g * L, L)
        uu = jnp.clip(u_v[sl], 0.0, 1.0)
        x = uu * jnp.float32(W - 1)
        idx = jnp.minimum(x.astype(jnp.int32), W - 2)
        t = x - idx.astype(jnp.float32)
        t2 = t * t
        t3 = t2 * t
        sixth = jnp.float32(1.0 / 6.0)
        w0 = sixth * (-t3 + 3.0 * t2 - 3.0 * t + 1.0)
        w1 = sixth * (3.0 * t3 - 6.0 * t2 + 4.0)
        w2 = 0.5 * (-t3 + t2 + t) + sixth
        w3 = sixth * t3
        is_lo = idx == 0
        is_hi = idx == W - 2
        v0 = jnp.where(is_lo, 2.0 * w0, w0)
        v1 = jnp.where(is_hi, w1 - w3, w1)
        v2 = w2 + jnp.where(is_lo, -w0, jnp.where(is_hi, w3, 0.0))
        v3 = w3
        idx_v[0, sl] = jnp.maximum(idx - 1, 0)
        idx_v[1, sl] = idx
        idx_v[2, sl] = idx + 1
        idx_v[3, sl] = jnp.minimum(idx + 2, W - 1)
        w_v[0, sl] = v0
        w_v[1, sl] = v1
        w_v[2, sl] = v2
        w_v[3, sl] = v3
        return carry

    sems = (sem0, sem1)

    def fire(c):
        return [
            pltpu.async_copy(
                grid_hbm.at[idx_v.at[k, pl.ds(c * CH, CH)]],
                rows_v.at[c % 2, k], sems[c % 2])
            for k in range(4)
        ]

    lax.fori_loop(0, CH // L, wcomp, 0, unroll=2)
    inflight = fire(0)
    lax.fori_loop(CH // L, bpw // L, wcomp, 0, unroll=2)
    for c in range(nch):
        for cp in inflight:
            cp.wait()
        if c + 1 < nch:
            inflight = fire(c + 1)

        def scomp(g2, carry, c=c):
            wsl = pl.ds(c * CH + g2 * L, L)
            a0 = w_v[0, wsl]
            a1 = w_v[1, wsl]
            a2 = w_v[2, wsl]
            a3 = w_v[3, wsl]
            for j in range(L):
                b = g2 * L + j
                s0, s1, s2, s3 = a0[j], a1[j], a2[j], a3[j]
                for jj in range(ND // L):
                    sl = pl.ds(jj * L, L)
                    acc = (rows_v[c % 2, 0, b, sl] * s0
                           + rows_v[c % 2, 1, b, sl] * s1
                           + rows_v[c % 2, 2, b, sl] * s2
                           + rows_v[c % 2, 3, b, sl] * s3)
                    out_v[b, sl] = acc
            return carry

        lax.fori_loop(0, CH // L, scomp, 0)
        pltpu.sync_copy(out_v, out_hbm.at[pl.ds(base + c * CH, CH)])


def kernel(u, grid):
    B = u.shape[0]
    W, ND = grid.shape
    bpw = B // NW
    nch = bpw // CH
    mesh = plsc.VectorSubcoreMesh(core_axis_name="c", subcore_axis_name="s")
    body = functools.partial(_spline_body, W, ND, bpw, nch)
    f = pl.kernel(
        body,
        mesh=mesh,
        out_type=jax.ShapeDtypeStruct((B, ND), jnp.float32),
        scratch_types=[
            pltpu.VMEM((bpw,), jnp.float32),
            pltpu.VMEM((4, bpw), jnp.int32),
            pltpu.VMEM((4, bpw), jnp.float32),
            pltpu.VMEM((2, 4, CH, ND), jnp.float32),
            pltpu.VMEM((CH, ND), jnp.float32),
            pltpu.SemaphoreType.DMA,
            pltpu.SemaphoreType.DMA,
        ],
        compiler_params=pltpu.CompilerParams(use_tc_tiling_on_sc=False),
    )
    return f(u, grid)
